# baseline (speedup 1.0000x reference)
import functools

import jax
import jax.numpy as jnp
from jax import lax
from jax.experimental import pallas as pl
from jax.experimental.pallas import tpu as pltpu
from jax.experimental.pallas import tpu_sc as plsc

_N = 10000
_D = 128
_E = 320000
_NC = 2
_NS = 16
_CHUNK = 128
_NCH = 80
_HALF = _NCH // 2
_EPAD = _NC * _NS * _NCH * _CHUNK
_RPT = 632
_NPAD = _NS * _RPT
_RB = 1000


def _make_segsum():
    mesh = plsc.VectorSubcoreMesh(core_axis_name="c", subcore_axis_name="s")

    @functools.partial(
        pl.kernel,
        out_type=jax.ShapeDtypeStruct((_NC, _NPAD, _D), jnp.float32),
        mesh=mesh,
        scratch_types=[
            pltpu.VMEM((_HALF, _CHUNK), jnp.int32),
            pltpu.VMEM((_HALF, _CHUNK), jnp.int32),
            pltpu.VMEM((_CHUNK, _D), jnp.float32),
            pltpu.VMEM((_CHUNK, _D), jnp.float32),
            pltpu.VMEM_SHARED((_NPAD, _D), jnp.float32),
            pltpu.SemaphoreType.DMA,
            pltpu.SemaphoreType.DMA,
            pltpu.SemaphoreType.DMA,
            pltpu.SemaphoreType.DMA,
        ],
    )
    def segsum(feat, srcs, dsts, zrows, out, src_v, dst_v, rows0, rows1,
               acc, gsem0, gsem1, ssem0, ssem1):
        c = lax.axis_index("c")
        s = lax.axis_index("s")
        pltpu.sync_copy(zrows, acc.at[pl.ds(s * _RPT, _RPT)])
        plsc.subcore_barrier()

        rows = (rows0, rows1)
        gsems = (gsem0, gsem1)
        ssems = (ssem0, ssem1)

        def gat(j, b):
            pltpu.async_copy(feat.at[src_v.at[j]], rows[b], gsems[b])

        def sca(j, b):
            pltpu.async_copy(rows[b], acc.at[dst_v.at[j]], ssems[b],
                             add=True)

        def gwait(b):
            pltpu.make_async_copy(feat.at[src_v.at[0]], rows[b],
                                  gsems[b]).wait()

        def swait(b):
            pltpu.make_async_copy(rows[b], acc.at[dst_v.at[0]],
                                  ssems[b]).wait()

        for h in range(2):
            pltpu.sync_copy(srcs.at[c, s, h], src_v)
            pltpu.sync_copy(dsts.at[c, s, h], dst_v)
            gat(0, 0)
            gat(1, 1)
            gwait(0)
            sca(0, 0)

            def pipe(i, carry):
                jj = 2 * i
                swait(0)
                gat(jj, 0)
                gwait(1)
                sca(jj - 1, 1)
                swait(1)
                gat(jj + 1, 1)
                gwait(0)
                sca(jj, 0)
                return carry

            lax.fori_loop(1, _HALF // 2, pipe, 0)

            gwait(1)
            sca(_HALF - 1, 1)
            swait(0)
            swait(1)

        plsc.subcore_barrier()
        pltpu.sync_copy(acc.at[pl.ds(s * _RPT, _RPT)],
                        out.at[c, pl.ds(s * _RPT, _RPT)])

    return segsum


def _fused_linear(p0, p1, wt, b, relu):
    def body(p0_ref, p1_ref, wt_ref, b_ref, o_ref):
        a = p0_ref[...] + p1_ref[...]
        y = jnp.dot(a, wt_ref[...], preferred_element_type=jnp.float32) + b_ref[...]
        if relu:
            y = jnp.maximum(y, 0.0)
        o_ref[...] = y

    return pl.pallas_call(
        body,
        grid=(_N // _RB,),
        in_specs=[
            pl.BlockSpec((_RB, _D), lambda i: (i, 0)),
            pl.BlockSpec((_RB, _D), lambda i: (i, 0)),
            pl.BlockSpec((_D, _D), lambda i: (0, 0)),
            pl.BlockSpec((1, _D), lambda i: (0, 0)),
        ],
        out_specs=pl.BlockSpec((_RB, _D), lambda i: (i, 0)),
        out_shape=jax.ShapeDtypeStruct((_N, _D), jnp.float32),
    )(p0, p1, wt, b)


def _proj_head(h2, p1t, pb1, p2t, pb2):
    def body(h_ref, p1t_ref, pb1_ref, p2t_ref, pb2_ref, z_ref):
        t = jnp.maximum(
            jnp.dot(h_ref[...], p1t_ref[...],
                    preferred_element_type=jnp.float32) + pb1_ref[...], 0.0)
        z_ref[...] = jnp.dot(t, p2t_ref[...],
                             preferred_element_type=jnp.float32) + pb2_ref[...]

    mat = lambda: pl.BlockSpec((_D, _D), lambda i: (0, 0))
    vec = lambda: pl.BlockSpec((1, _D), lambda i: (0, 0))
    row = lambda: pl.BlockSpec((_RB, _D), lambda i: (i, 0))
    return pl.pallas_call(
        body,
        grid=(_N // _RB,),
        in_specs=[row(), mat(), vec(), mat(), vec()],
        out_specs=row(),
        out_shape=jax.ShapeDtypeStruct((_N, _D), jnp.float32),
    )(h2, p1t, pb1, p2t, pb2)


def kernel(x, edge_index, W1, b1, bn1_g, bn1_b, bn1_m, bn1_v, W2, b2,
           bn2_g, bn2_b, bn2_m, bn2_v, W3, b3, P1, pb1, P2, pb2):
    eps = 1e-5
    s1 = bn1_g * lax.rsqrt(bn1_v + eps)
    w1t = (W1 * s1[:, None]).T
    b1e = ((b1 - bn1_m) * s1 + bn1_b)[None, :]
    s2 = bn2_g * lax.rsqrt(bn2_v + eps)
    w2t = (W2 * s2[:, None]).T
    b2e = ((b2 - bn2_m) * s2 + bn2_b)[None, :]

    srcp = jnp.pad(edge_index[0], (0, _EPAD - _E)).reshape(
        _NC, _NS, 2, _HALF, _CHUNK)
    dstp = jnp.pad(edge_index[1], (0, _EPAD - _E),
                   constant_values=_N).reshape(_NC, _NS, 2, _HALF, _CHUNK)
    zrows = jnp.zeros((_RPT, _D), jnp.float32)

    segsum = _make_segsum()
    agg1 = segsum(x, srcp, dstp, zrows)
    h1 = _fused_linear(agg1[0, :_N], agg1[1, :_N], w1t, b1e, True)
    agg2 = segsum(h1, srcp, dstp, zrows)
    h2 = _fused_linear(agg2[0, :_N], agg2[1, :_N], w2t, b2e, True)
    z = _proj_head(h2, P1.T, pb1[None, :], P2.T, pb2[None, :])
    agg3 = segsum(h2, srcp, dstp, zrows)
    logits = _fused_linear(agg3[0, :_N], agg3[1, :_N], W3.T, b3[None, :], False)
    return (logits, h2, z)


# speedup vs baseline: 3.5589x; 3.5589x over previous
import functools

import jax
import jax.numpy as jnp
from jax import lax
from jax.experimental import pallas as pl
from jax.experimental.pallas import tpu as pltpu
from jax.experimental.pallas import tpu_sc as plsc

_N = 10000
_D = 128
_E = 320000
_NC = 2
_NS = 16
_CHUNK = 128
_NCH = 80
_HALF = _NCH // 2
_EPAD = _NC * _NS * _NCH * _CHUNK
_RPT = 632
_NPAD = _NS * _RPT
_RB = 1000


def _make_segsum():
    mesh = plsc.VectorSubcoreMesh(core_axis_name="c", subcore_axis_name="s")

    @functools.partial(
        pl.kernel,
        out_type=jax.ShapeDtypeStruct((_NC, _NPAD, _D), jnp.float32),
        mesh=mesh,
        scratch_types=[
            pltpu.VMEM((_HALF, _CHUNK), jnp.int32),
            pltpu.VMEM((_HALF, _CHUNK), jnp.int32),
            pltpu.VMEM((_CHUNK, _D), jnp.float32),
            pltpu.VMEM((_CHUNK, _D), jnp.float32),
            pltpu.VMEM_SHARED((_NPAD, _D), jnp.float32),
            pltpu.SemaphoreType.DMA,
            pltpu.SemaphoreType.DMA,
            pltpu.SemaphoreType.DMA,
            pltpu.SemaphoreType.DMA,
        ],
    )
    def segsum(feat, srcs, dsts, zrows, out, src_v, dst_v, rows0, rows1,
               acc, gsem0, gsem1, ssem0, ssem1):
        c = lax.axis_index("c")
        s = lax.axis_index("s")
        pltpu.sync_copy(zrows, acc.at[pl.ds(s * _RPT, _RPT)])
        plsc.subcore_barrier()

        rows = (rows0, rows1)
        gsems = (gsem0, gsem1)
        ssems = (ssem0, ssem1)

        def gat(j, b):
            pltpu.async_copy(feat.at[src_v.at[j]], rows[b], gsems[b])

        def sca(j, b):
            pltpu.async_copy(rows[b], acc.at[dst_v.at[j]], ssems[b],
                             add=True)

        def gwait(b):
            pltpu.make_async_copy(feat.at[src_v.at[0]], rows[b],
                                  gsems[b]).wait()

        def swait(b):
            pltpu.make_async_copy(rows[b], acc.at[dst_v.at[0]],
                                  ssems[b]).wait()

        for h in range(2):
            pltpu.sync_copy(srcs.at[c, s, h], src_v)
            pltpu.sync_copy(dsts.at[c, s, h], dst_v)
            gat(0, 0)
            gat(1, 1)
            gwait(0)
            sca(0, 0)

            def pipe(i, carry):
                jj = 2 * i
                swait(0)
                gat(jj, 0)
                gwait(1)
                sca(jj - 1, 1)
                swait(1)
                gat(jj + 1, 1)
                gwait(0)
                sca(jj, 0)
                return carry

            lax.fori_loop(1, _HALF // 2, pipe, 0)

            gwait(1)
            sca(_HALF - 1, 1)
            swait(0)
            swait(1)

        plsc.subcore_barrier()
        pltpu.sync_copy(acc.at[pl.ds(s * _RPT, _RPT)],
                        out.at[c, pl.ds(s * _RPT, _RPT)])

    return segsum


def _fused_linear(p0, p1, wt, b, relu):
    def body(p0_ref, p1_ref, wt_ref, b_ref, o_ref):
        a = p0_ref[...] + p1_ref[...]
        y = jnp.dot(a, wt_ref[...], preferred_element_type=jnp.float32) + b_ref[...]
        if relu:
            y = jnp.maximum(y, 0.0)
        o_ref[...] = y

    return pl.pallas_call(
        body,
        grid=(_N // _RB,),
        in_specs=[
            pl.BlockSpec((_RB, _D), lambda i: (i, 0)),
            pl.BlockSpec((_RB, _D), lambda i: (i, 0)),
            pl.BlockSpec((_D, _D), lambda i: (0, 0)),
            pl.BlockSpec((1, _D), lambda i: (0, 0)),
        ],
        out_specs=pl.BlockSpec((_RB, _D), lambda i: (i, 0)),
        out_shape=jax.ShapeDtypeStruct((_N, _D), jnp.float32),
    )(p0, p1, wt, b)


def _proj_head(h2, p1t, pb1, p2t, pb2):
    def body(h_ref, p1t_ref, pb1_ref, p2t_ref, pb2_ref, z_ref):
        t = jnp.maximum(
            jnp.dot(h_ref[...], p1t_ref[...],
                    preferred_element_type=jnp.float32) + pb1_ref[...], 0.0)
        z_ref[...] = jnp.dot(t, p2t_ref[...],
                             preferred_element_type=jnp.float32) + pb2_ref[...]

    mat = lambda: pl.BlockSpec((_D, _D), lambda i: (0, 0))
    vec = lambda: pl.BlockSpec((1, _D), lambda i: (0, 0))
    row = lambda: pl.BlockSpec((_RB, _D), lambda i: (i, 0))
    return pl.pallas_call(
        body,
        grid=(_N // _RB,),
        in_specs=[row(), mat(), vec(), mat(), vec()],
        out_specs=row(),
        out_shape=jax.ShapeDtypeStruct((_N, _D), jnp.float32),
    )(h2, p1t, pb1, p2t, pb2)


def kernel(x, edge_index, W1, b1, bn1_g, bn1_b, bn1_m, bn1_v, W2, b2,
           bn2_g, bn2_b, bn2_m, bn2_v, W3, b3, P1, pb1, P2, pb2):
    eps = 1e-5
    s1 = bn1_g * lax.rsqrt(bn1_v + eps)
    w1t = (W1 * s1[:, None]).T
    b1e = ((b1 - bn1_m) * s1 + bn1_b)[None, :]
    s2 = bn2_g * lax.rsqrt(bn2_v + eps)
    w2t = (W2 * s2[:, None]).T
    b2e = ((b2 - bn2_m) * s2 + bn2_b)[None, :]

    pad_ar = jnp.arange(_EPAD - _E, dtype=jnp.int32)
    srcp = jnp.concatenate([edge_index[0], pad_ar % _N]).reshape(
        _NC, _NS, 2, _HALF, _CHUNK)
    dstp = jnp.concatenate([edge_index[1], _N + pad_ar % (_NPAD - _N)]).reshape(
        _NC, _NS, 2, _HALF, _CHUNK)
    zrows = jnp.zeros((_RPT, _D), jnp.float32)

    segsum = _make_segsum()
    agg1 = segsum(x, srcp, dstp, zrows)
    h1 = _fused_linear(agg1[0, :_N], agg1[1, :_N], w1t, b1e, True)
    agg2 = segsum(h1, srcp, dstp, zrows)
    h2 = _fused_linear(agg2[0, :_N], agg2[1, :_N], w2t, b2e, True)
    z = _proj_head(h2, P1.T, pb1[None, :], P2.T, pb2[None, :])
    agg3 = segsum(h2, srcp, dstp, zrows)
    logits = _fused_linear(agg3[0, :_N], agg3[1, :_N], W3.T, b3[None, :], False)
    return (logits, h2, z)


# speedup vs baseline: 3.5660x; 1.0020x over previous
import functools

import jax
import jax.numpy as jnp
from jax import lax
from jax.experimental import pallas as pl
from jax.experimental.pallas import tpu as pltpu
from jax.experimental.pallas import tpu_sc as plsc

_N = 10000
_D = 128
_E = 320000
_NC = 2
_NS = 16
_CHUNK = 128
_NCH = 80
_HALF = _NCH // 2
_EPAD = _NC * _NS * _NCH * _CHUNK
_RPT = 632
_NPAD = _NS * _RPT
_RB = 1000


def _make_segsum():
    mesh = plsc.VectorSubcoreMesh(core_axis_name="c", subcore_axis_name="s")

    @functools.partial(
        pl.kernel,
        out_type=jax.ShapeDtypeStruct((_NC, _NPAD, _D), jnp.float32),
        mesh=mesh,
        scratch_types=[
            pltpu.VMEM((_HALF, _CHUNK), jnp.int32),
            pltpu.VMEM((_HALF, _CHUNK), jnp.int32),
            pltpu.VMEM((_CHUNK, _D), jnp.float32),
            pltpu.VMEM((_CHUNK, _D), jnp.float32),
            pltpu.VMEM_SHARED((_NPAD, _D), jnp.float32),
            pltpu.SemaphoreType.DMA,
            pltpu.SemaphoreType.DMA,
            pltpu.SemaphoreType.DMA,
            pltpu.SemaphoreType.DMA,
        ],
    )
    def segsum(feat, srcs, dsts, zrows, out, src_v, dst_v, rows0, rows1,
               acc, gsem0, gsem1, ssem0, ssem1):
        c = lax.axis_index("c")
        s = lax.axis_index("s")
        pltpu.sync_copy(zrows, acc.at[pl.ds(s * _RPT, _RPT)])
        plsc.subcore_barrier()

        rows = (rows0, rows1)
        gsems = (gsem0, gsem1)
        ssems = (ssem0, ssem1)

        def gat(j, b):
            pltpu.async_copy(feat.at[src_v.at[j]], rows[b], gsems[b])

        def sca(j, b):
            pltpu.async_copy(rows[b], acc.at[dst_v.at[j]], ssems[b],
                             add=True)

        def gwait(b):
            pltpu.make_async_copy(feat.at[src_v.at[0]], rows[b],
                                  gsems[b]).wait()

        def swait(b):
            pltpu.make_async_copy(rows[b], acc.at[dst_v.at[0]],
                                  ssems[b]).wait()

        for h in range(2):
            pltpu.sync_copy(srcs.at[c, s, h], src_v)
            pltpu.sync_copy(dsts.at[c, s, h], dst_v)
            gat(0, 0)
            gat(1, 1)
            gwait(0)
            sca(0, 0)

            def pipe(i, carry):
                jj = 2 * i
                swait(0)
                gat(jj, 0)
                gwait(1)
                sca(jj - 1, 1)
                swait(1)
                gat(jj + 1, 1)
                gwait(0)
                sca(jj, 0)
                return carry

            lax.fori_loop(1, _HALF // 2, pipe, 0, unroll=2)

            gwait(1)
            sca(_HALF - 1, 1)
            swait(0)
            swait(1)

        plsc.subcore_barrier()
        pltpu.sync_copy(acc.at[pl.ds(s * _RPT, _RPT)],
                        out.at[c, pl.ds(s * _RPT, _RPT)])

    return segsum


def _fused_linear(p0, p1, wt, b, relu):
    def body(p0_ref, p1_ref, wt_ref, b_ref, o_ref):
        a = p0_ref[...] + p1_ref[...]
        y = jnp.dot(a, wt_ref[...], preferred_element_type=jnp.float32) + b_ref[...]
        if relu:
            y = jnp.maximum(y, 0.0)
        o_ref[...] = y

    return pl.pallas_call(
        body,
        grid=(_N // _RB,),
        in_specs=[
            pl.BlockSpec((_RB, _D), lambda i: (i, 0)),
            pl.BlockSpec((_RB, _D), lambda i: (i, 0)),
            pl.BlockSpec((_D, _D), lambda i: (0, 0)),
            pl.BlockSpec((1, _D), lambda i: (0, 0)),
        ],
        out_specs=pl.BlockSpec((_RB, _D), lambda i: (i, 0)),
        out_shape=jax.ShapeDtypeStruct((_N, _D), jnp.float32),
    )(p0, p1, wt, b)


def _proj_head(h2, p1t, pb1, p2t, pb2):
    def body(h_ref, p1t_ref, pb1_ref, p2t_ref, pb2_ref, z_ref):
        t = jnp.maximum(
            jnp.dot(h_ref[...], p1t_ref[...],
                    preferred_element_type=jnp.float32) + pb1_ref[...], 0.0)
        z_ref[...] = jnp.dot(t, p2t_ref[...],
                             preferred_element_type=jnp.float32) + pb2_ref[...]

    mat = lambda: pl.BlockSpec((_D, _D), lambda i: (0, 0))
    vec = lambda: pl.BlockSpec((1, _D), lambda i: (0, 0))
    row = lambda: pl.BlockSpec((_RB, _D), lambda i: (i, 0))
    return pl.pallas_call(
        body,
        grid=(_N // _RB,),
        in_specs=[row(), mat(), vec(), mat(), vec()],
        out_specs=row(),
        out_shape=jax.ShapeDtypeStruct((_N, _D), jnp.float32),
    )(h2, p1t, pb1, p2t, pb2)


def kernel(x, edge_index, W1, b1, bn1_g, bn1_b, bn1_m, bn1_v, W2, b2,
           bn2_g, bn2_b, bn2_m, bn2_v, W3, b3, P1, pb1, P2, pb2):
    eps = 1e-5
    s1 = bn1_g * lax.rsqrt(bn1_v + eps)
    w1t = (W1 * s1[:, None]).T
    b1e = ((b1 - bn1_m) * s1 + bn1_b)[None, :]
    s2 = bn2_g * lax.rsqrt(bn2_v + eps)
    w2t = (W2 * s2[:, None]).T
    b2e = ((b2 - bn2_m) * s2 + bn2_b)[None, :]

    pad_ar = jnp.arange(_EPAD - _E, dtype=jnp.int32)
    srcp = jnp.concatenate([edge_index[0], pad_ar % _N]).reshape(
        _NC, _NS, 2, _HALF, _CHUNK)
    dstp = jnp.concatenate([edge_index[1], _N + pad_ar % (_NPAD - _N)]).reshape(
        _NC, _NS, 2, _HALF, _CHUNK)
    zrows = jnp.zeros((_RPT, _D), jnp.float32)

    segsum = _make_segsum()
    agg1 = segsum(x, srcp, dstp, zrows)
    h1 = _fused_linear(agg1[0, :_N], agg1[1, :_N], w1t, b1e, True)
    agg2 = segsum(h1, srcp, dstp, zrows)
    h2 = _fused_linear(agg2[0, :_N], agg2[1, :_N], w2t, b2e, True)
    z = _proj_head(h2, P1.T, pb1[None, :], P2.T, pb2[None, :])
    agg3 = segsum(h2, srcp, dstp, zrows)
    logits = _fused_linear(agg3[0, :_N], agg3[1, :_N], W3.T, b3[None, :], False)
    return (logits, h2, z)
